# baseline (speedup 1.0000x reference)
import jax
import jax.numpy as jnp
from jax.experimental import pallas as pl
from jax.experimental.pallas import tpu as pltpu

_K = 16
_RATIO = 3.0
_MAXBITS = 0x7F7FFFFF


def _ce_pair(l0, l1, lbl_is1):
    m = jnp.maximum(l0, l1)
    lse = m + jnp.log(jnp.exp(l0 - m) + jnp.exp(l1 - m))
    return lse - jnp.where(lbl_is1, l1, l0)


def _smooth_l1(t, p):
    ad = jnp.abs(t - p)
    m = jnp.minimum(ad, 1.0)
    return m * (ad - 0.5 * m)


def kernel(cls_p3, reg_p3, cls_p4, reg_p4, cls_p5, reg_p5, gt_p3, gt_p4, gt_p5):
    n_batch = cls_p3.shape[0]
    ins = [cls_p3, reg_p3, gt_p3, cls_p4, reg_p4, gt_p4, cls_p5, reg_p5, gt_p5]
    hs = [cls_p3.shape[2], cls_p4.shape[2], cls_p5.shape[2]]
    ms = [float(n_batch * h * h) for h in hs]

    def body(cls3, reg3, gt3, cls4, reg4, gt4, cls5, reg5, gt5,
             out_ref, ce3, ce4, ce5, acc3, acc4, acc5):
        n = pl.program_id(0)

        def accum_level(cls_r, reg_r, gt_r, ce_r, acc, nloc):
            trm = gt_r[0, 0]
            tclm = gt_r[0, 1]
            trainm = gt_r[0, 2]
            pos = (trm * trainm) > 0
            neg = (trainm - trm * trainm) > 0
            ce_tr = _ce_pair(cls_r[0, 0], cls_r[0, 1], trm > 0)
            bits = jax.lax.bitcast_convert_type(ce_tr, jnp.int32)
            ce_r[pl.ds(nloc, 1)] = jnp.where(neg, bits, jnp.int32(-1))[None]
            tr_train = pos
            ce_tcl = _ce_pair(cls_r[0, 2], cls_r[0, 3], tclm > 0)
            wmask = jnp.where(tr_train, (trm + tclm) * 0.5, 0.0)
            csx = jnp.sum(_smooth_l1(gt_r[0, 3:3 + _K], reg_r[0, 0:_K]),
                          axis=0)
            csy = jnp.sum(_smooth_l1(gt_r[0, 3 + _K:3 + 2 * _K],
                                     reg_r[0, _K:2 * _K]), axis=0)
            ce_neg = jnp.where(neg, ce_tr, -1.0)
            ce_neg_inf = jnp.where(neg, ce_tr, jnp.inf)
            planes = (
                pos.astype(jnp.float32),
                neg.astype(jnp.float32),
                jnp.where(pos, ce_tr, 0.0),
                jnp.where(tr_train, ce_tcl, 0.0),
                jnp.where(tr_train, 0.0, ce_tcl),
                wmask * csx,
                wmask * csy,
            )

            @pl.when(nloc == 0)
            def _store():
                for i, q in enumerate(planes):
                    acc[i] = q
                acc[7] = ce_neg
                acc[8] = ce_neg_inf

            @pl.when(nloc > 0)
            def _accum():
                for i, q in enumerate(planes):
                    acc[i] += q
                acc[7] = jnp.maximum(acc[7], ce_neg)
                acc[8] = jnp.minimum(acc[8], ce_neg_inf)

        accum_level(cls3, reg3, gt3, ce3, acc3, n)
        accum_level(cls4, reg4, gt4, ce4, acc4, n)

        @pl.when(jax.lax.rem(n, 2) == 0)
        def _l5():
            accum_level(cls5, reg5, gt5, ce5, acc5, jax.lax.div(n, 2))

        @pl.when(n == 2 * n_batch - 1)
        def _final():
            ks = []
            nnegs = []
            stats = []
            for acc in (acc3, acc4, acc5):
                n_pos = jnp.sum(acc[0])
                neg_count = jnp.sum(acc[1])
                stats.append((n_pos, neg_count, jnp.sum(acc[2]),
                              jnp.sum(acc[3]), jnp.sum(acc[4]),
                              jnp.sum(acc[5]), jnp.sum(acc[6])))
            for j in range(3):
                n_pos = stats[j][0]
                neg_count = stats[j][1]
                n_neg = jnp.where(
                    n_pos > 0,
                    jnp.minimum(neg_count, jnp.floor(_RATIO * n_pos)), 100.0)
                nnegs.append(n_neg)
                ks.append(jnp.minimum(n_neg, neg_count))

            b3 = ce3[...]
            b4 = ce4[...]
            b5 = ce5[...]

            bounds = []
            for j, acc in enumerate((acc3, acc4, acc5)):
                mx = jax.lax.bitcast_convert_type(jnp.max(acc[7]), jnp.int32)
                mn = jax.lax.bitcast_convert_type(jnp.min(acc[8]), jnp.int32)
                skip = ks[j] >= stats[j][1]
                z = jnp.int32(0)
                bounds.append(
                    (jnp.where(skip, z,
                               jnp.minimum(jnp.maximum(mn, 0), _MAXBITS)),
                     jnp.where(skip, z, jnp.minimum(mx, _MAXBITS))))

            def search_step2(b, lo, hi, k):
                sr = jax.lax.shift_right_logical
                mid = lo + sr(hi - lo + 1, 1)
                mid_a = mid + sr(hi - mid + 1, 1)
                mid_b = lo + sr(mid - lo, 1)
                c1 = jnp.sum((b >= mid).astype(jnp.float32))
                c2a = jnp.sum((b >= mid_a).astype(jnp.float32))
                c2b = jnp.sum((b >= mid_b).astype(jnp.float32))
                ge1 = c1 >= k
                lo1 = jnp.where(ge1, mid, lo)
                hi1 = jnp.where(ge1, hi, mid - 1)
                mid2 = jnp.where(ge1, mid_a, mid_b)
                ge2 = jnp.where(ge1, c2a, c2b) >= k
                return (jnp.where(ge2, mid2, lo1),
                        jnp.where(ge2, hi1, mid2 - 1))

            def w_cond(st):
                lo3, hi3, lo4, hi4, lo5, hi5 = st
                return (lo3 < hi3) | (lo4 < hi4) | (lo5 < hi5)

            def w_body(st):
                lo3, hi3, lo4, hi4, lo5, hi5 = st
                lo3, hi3 = search_step2(b3, lo3, hi3, ks[0])
                lo4, hi4 = search_step2(b4, lo4, hi4, ks[1])
                lo5, hi5 = search_step2(b5, lo5, hi5, ks[2])
                return lo3, hi3, lo4, hi4, lo5, hi5

            st = jax.lax.while_loop(
                w_cond, w_body,
                (bounds[0][0], bounds[0][1], bounds[1][0], bounds[1][1],
                 bounds[2][0], bounds[2][1]))
            thresholds = (st[0], st[2], st[4])

            loss_text = 0.0
            loss_center = 0.0
            loss_rx = 0.0
            loss_ry = 0.0
            for j, b in enumerate((b3, b4, b5)):
                t = thresholds[j]
                gt_m = b > t
                cnt_gt = jnp.sum(gt_m.astype(jnp.float32))
                vals = jax.lax.bitcast_convert_type(b, jnp.float32)
                sum_gt = jnp.sum(jnp.where(gt_m, vals, 0.0))
                tval = jax.lax.bitcast_convert_type(t, jnp.float32)
                k = ks[j]
                loss_neg = jnp.where(k > 0, sum_gt + (k - cnt_gt) * tval, 0.0)
                n_pos = stats[j][0]
                loss_pos = jnp.where(n_pos > 0, stats[j][2], 0.0)
                loss_text += (loss_pos + loss_neg) / (n_pos + nnegs[j])
                p_count = n_pos
                ng_count = ms[j] - p_count
                loss_center += jnp.where(
                    p_count > 0,
                    stats[j][3] / p_count + 0.5 * stats[j][4] / ng_count, 0.0)
                loss_rx += jnp.where(p_count > 0,
                                     stats[j][5] / (p_count * _K), 0.0)
                loss_ry += jnp.where(p_count > 0,
                                     stats[j][6] / (p_count * _K), 0.0)
            out_ref[0] = loss_text
            out_ref[1] = loss_center
            out_ref[2] = loss_rx
            out_ref[3] = loss_ry

    def _imap_half(n):
        return (jax.lax.div(n, 2), 0, jax.lax.rem(n, 2), 0)

    def _imap_full(n):
        return (jax.lax.div(n, 2), 0, 0, 0)

    specs = []
    for i, a in enumerate(ins):
        if i < 6:
            specs.append(pl.BlockSpec(
                (1, a.shape[1], a.shape[2] // 2, a.shape[3]), _imap_half))
        else:
            specs.append(pl.BlockSpec((1,) + a.shape[1:], _imap_full))
    out = pl.pallas_call(
        body,
        grid=(2 * n_batch,),
        in_specs=specs,
        out_specs=pl.BlockSpec(memory_space=pltpu.SMEM),
        out_shape=jax.ShapeDtypeStruct((4,), jnp.float32),
        scratch_shapes=[
            pltpu.VMEM((2 * n_batch, hs[0] // 2, hs[0]), jnp.int32),
            pltpu.VMEM((2 * n_batch, hs[1] // 2, hs[1]), jnp.int32),
            pltpu.VMEM((n_batch, hs[2], hs[2]), jnp.int32),
            pltpu.VMEM((9, hs[0] // 2, hs[0]), jnp.float32),
            pltpu.VMEM((9, hs[1] // 2, hs[1]), jnp.float32),
            pltpu.VMEM((9, hs[2], hs[2]), jnp.float32),
        ],
    )(*ins)
    return (out[0], out[1], out[2], out[3])


# speedup vs baseline: 1.0478x; 1.0478x over previous
import jax
import jax.numpy as jnp
from jax.experimental import pallas as pl
from jax.experimental.pallas import tpu as pltpu

_K = 16
_RATIO = 3.0
_MAXBITS = 0x7F7FFFFF


def _ce_pair(l0, l1, lbl_is1):
    m = jnp.maximum(l0, l1)
    lse = m + jnp.log(jnp.exp(l0 - m) + jnp.exp(l1 - m))
    return lse - jnp.where(lbl_is1, l1, l0)


def _smooth_l1(t, p):
    ad = jnp.abs(t - p)
    m = jnp.minimum(ad, 1.0)
    return m * (ad - 0.5 * m)


def kernel(cls_p3, reg_p3, cls_p4, reg_p4, cls_p5, reg_p5, gt_p3, gt_p4, gt_p5):
    n_batch = cls_p3.shape[0]
    ins = [cls_p3, reg_p3, gt_p3, cls_p4, reg_p4, gt_p4, cls_p5, reg_p5, gt_p5]
    hs = [cls_p3.shape[2], cls_p4.shape[2], cls_p5.shape[2]]
    ms = [float(n_batch * h * h) for h in hs]

    def body(cls3, reg3, gt3, cls4, reg4, gt4, cls5, reg5, gt5,
             out_ref, ce3, ce4, ce5, acc3, acc4, acc5):
        n = pl.program_id(0)

        for j, (cls_r, reg_r, gt_r, ce_r, acc) in enumerate(
                ((cls3, reg3, gt3, ce3, acc3), (cls4, reg4, gt4, ce4, acc4),
                 (cls5, reg5, gt5, ce5, acc5))):
            trm = gt_r[0, 0]
            tclm = gt_r[0, 1]
            trainm = gt_r[0, 2]
            pos = (trm * trainm) > 0
            neg = (trainm - trm * trainm) > 0
            ce_tr = _ce_pair(cls_r[0, 0], cls_r[0, 1], trm > 0)
            bits = jax.lax.bitcast_convert_type(ce_tr, jnp.int32)
            ce_r[pl.ds(n, 1)] = jnp.where(neg, bits, jnp.int32(-1))[None]
            tr_train = pos
            ce_tcl = _ce_pair(cls_r[0, 2], cls_r[0, 3], tclm > 0)
            wmask = jnp.where(tr_train, (trm + tclm) * 0.5, 0.0)
            csx = jnp.sum(_smooth_l1(gt_r[0, 3:3 + _K], reg_r[0, 0:_K]),
                          axis=0)
            csy = jnp.sum(_smooth_l1(gt_r[0, 3 + _K:3 + 2 * _K],
                                     reg_r[0, _K:2 * _K]), axis=0)
            ce_neg = jnp.where(neg, ce_tr, -1.0)
            ce_neg_inf = jnp.where(neg, ce_tr, jnp.inf)
            planes = (
                pos.astype(jnp.float32),
                neg.astype(jnp.float32),
                jnp.where(pos, ce_tr, 0.0),
                jnp.where(tr_train, ce_tcl, 0.0),
                jnp.where(tr_train, 0.0, ce_tcl),
                wmask * csx,
                wmask * csy,
            )

            @pl.when(n == 0)
            def _store():
                for i, q in enumerate(planes):
                    acc[i] = q
                acc[7] = ce_neg
                acc[8] = ce_neg_inf

            @pl.when(n > 0)
            def _accum():
                for i, q in enumerate(planes):
                    acc[i] += q
                acc[7] = jnp.maximum(acc[7], ce_neg)
                acc[8] = jnp.minimum(acc[8], ce_neg_inf)

        @pl.when(n == n_batch - 1)
        def _final():
            ks = []
            nnegs = []
            stats = []
            for acc in (acc3, acc4, acc5):
                n_pos = jnp.sum(acc[0])
                neg_count = jnp.sum(acc[1])
                stats.append((n_pos, neg_count, jnp.sum(acc[2]),
                              jnp.sum(acc[3]), jnp.sum(acc[4]),
                              jnp.sum(acc[5]), jnp.sum(acc[6])))
            for j in range(3):
                n_pos = stats[j][0]
                neg_count = stats[j][1]
                n_neg = jnp.where(
                    n_pos > 0,
                    jnp.minimum(neg_count, jnp.floor(_RATIO * n_pos)), 100.0)
                nnegs.append(n_neg)
                ks.append(jnp.minimum(n_neg, neg_count))

            b3 = ce3[...]
            b4 = ce4[...]
            b5 = ce5[...]

            bounds = []
            for j, acc in enumerate((acc3, acc4, acc5)):
                mx = jax.lax.bitcast_convert_type(jnp.max(acc[7]), jnp.int32)
                mn = jax.lax.bitcast_convert_type(jnp.min(acc[8]), jnp.int32)
                skip = ks[j] >= stats[j][1]
                z = jnp.int32(0)
                bounds.append(
                    (jnp.where(skip, z,
                               jnp.minimum(jnp.maximum(mn, 0), _MAXBITS)),
                     jnp.where(skip, z, jnp.minimum(mx, _MAXBITS))))

            def search_step2(b, lo, hi, k):
                sr = jax.lax.shift_right_logical
                mid = lo + sr(hi - lo + 1, 1)
                mid_a = mid + sr(hi - mid + 1, 1)
                mid_b = lo + sr(mid - lo, 1)
                c1 = jnp.sum((b >= mid).astype(jnp.float32))
                c2a = jnp.sum((b >= mid_a).astype(jnp.float32))
                c2b = jnp.sum((b >= mid_b).astype(jnp.float32))
                ge1 = c1 >= k
                lo1 = jnp.where(ge1, mid, lo)
                hi1 = jnp.where(ge1, hi, mid - 1)
                mid2 = jnp.where(ge1, mid_a, mid_b)
                ge2 = jnp.where(ge1, c2a, c2b) >= k
                return (jnp.where(ge2, mid2, lo1),
                        jnp.where(ge2, hi1, mid2 - 1))

            def w_cond(st):
                lo3, hi3, lo4, hi4, lo5, hi5 = st
                return (lo3 < hi3) | (lo4 < hi4) | (lo5 < hi5)

            def w_body(st):
                lo3, hi3, lo4, hi4, lo5, hi5 = st
                lo3, hi3 = search_step2(b3, lo3, hi3, ks[0])
                lo4, hi4 = search_step2(b4, lo4, hi4, ks[1])
                lo5, hi5 = search_step2(b5, lo5, hi5, ks[2])
                return lo3, hi3, lo4, hi4, lo5, hi5

            st = jax.lax.while_loop(
                w_cond, w_body,
                (bounds[0][0], bounds[0][1], bounds[1][0], bounds[1][1],
                 bounds[2][0], bounds[2][1]))
            thresholds = (st[0], st[2], st[4])

            loss_text = 0.0
            loss_center = 0.0
            loss_rx = 0.0
            loss_ry = 0.0
            for j, b in enumerate((b3, b4, b5)):
                t = thresholds[j]
                gt_m = b > t
                cnt_gt = jnp.sum(gt_m.astype(jnp.float32))
                vals = jax.lax.bitcast_convert_type(b, jnp.float32)
                sum_gt = jnp.sum(jnp.where(gt_m, vals, 0.0))
                tval = jax.lax.bitcast_convert_type(t, jnp.float32)
                k = ks[j]
                loss_neg = jnp.where(k > 0, sum_gt + (k - cnt_gt) * tval, 0.0)
                n_pos = stats[j][0]
                loss_pos = jnp.where(n_pos > 0, stats[j][2], 0.0)
                loss_text += (loss_pos + loss_neg) / (n_pos + nnegs[j])
                p_count = n_pos
                ng_count = ms[j] - p_count
                loss_center += jnp.where(
                    p_count > 0,
                    stats[j][3] / p_count + 0.5 * stats[j][4] / ng_count, 0.0)
                loss_rx += jnp.where(p_count > 0,
                                     stats[j][5] / (p_count * _K), 0.0)
                loss_ry += jnp.where(p_count > 0,
                                     stats[j][6] / (p_count * _K), 0.0)
            out_ref[0] = loss_text
            out_ref[1] = loss_center
            out_ref[2] = loss_rx
            out_ref[3] = loss_ry

    specs = [pl.BlockSpec((1,) + a.shape[1:], lambda n: (n, 0, 0, 0))
             for a in ins]
    out = pl.pallas_call(
        body,
        grid=(n_batch,),
        in_specs=specs,
        out_specs=pl.BlockSpec(memory_space=pltpu.SMEM),
        out_shape=jax.ShapeDtypeStruct((4,), jnp.float32),
        scratch_shapes=[
            pltpu.VMEM((n_batch, hs[0], hs[0]), jnp.int32),
            pltpu.VMEM((n_batch, hs[1], hs[1]), jnp.int32),
            pltpu.VMEM((n_batch, hs[2], hs[2]), jnp.int32),
            pltpu.VMEM((9, hs[0], hs[0]), jnp.float32),
            pltpu.VMEM((9, hs[1], hs[1]), jnp.float32),
            pltpu.VMEM((9, hs[2], hs[2]), jnp.float32),
        ],
    )(*ins)
    return (out[0], out[1], out[2], out[3])
